# baseline (speedup 1.0000x reference)
import functools

import jax
import jax.numpy as jnp
from jax import lax
from jax.experimental import pallas as pl
from jax.experimental.pallas import tpu as pltpu
from jax.experimental.pallas import tpu_sc as plsc

B = 32
C = 384
H = 56
W = 56
NROW = B * H * W
NW = 32
RPW = NROW // NW
RCHUNK = 16
NCHUNK = RPW // RCHUNK
DEPTH = 4
L = 16
NJ = C // L


def _sel_body(in_hbm, ind_hbm, out_hbm, ind_v, idx_v,
              gin0, gin1, gin2, gin3, gout0, gout1, gout2, gout3,
              isem0, isem1, isem2, isem3, osem0, osem1, osem2, osem3):
    cid = lax.axis_index("c")
    sid = lax.axis_index("s")
    wid = sid * 2 + cid
    row0 = wid * RPW

    pltpu.sync_copy(ind_hbm, ind_v)

    zero_vec = jnp.zeros((L,), dtype=jnp.int32)
    for g in range(NJ):
        idx_v[pl.ds(g * L, L)] = zero_vec

    lane = lax.iota(jnp.int32, L)
    off = jnp.int32(0)
    for g in range(NJ):
        v = ind_v[pl.ds(g * L, L)]
        m = v != 0.0
        ind32 = jnp.where(m, 1, 0).astype(jnp.int32)
        cs = jnp.cumsum(ind32)
        pos = jnp.full((L,), off, dtype=jnp.int32) + cs - ind32
        vals = lane + (g * L)
        plsc.store_scatter(idx_v, [pos], vals, mask=m)
        off = off + jnp.max(cs)

    sel = [idx_v[pl.ds(j * L, L)] for j in range(NJ)]

    gins = (gin0, gin1, gin2, gin3)
    gouts = (gout0, gout1, gout2, gout3)
    isems = (isem0, isem1, isem2, isem3)
    osems = (osem0, osem1, osem2, osem3)

    def permute_chunk(gin, gout):
        @plsc.parallel_loop(0, RCHUNK, step=1, unroll=2)
        def body(r):
            row_splat = jnp.full((L,), r, dtype=jnp.int32)
            for j in range(NJ):
                vals = plsc.load_gather(gin, [row_splat, sel[j]])
                gout[r, pl.ds(j * L, L)] = vals

    def wait_in(p):
        pltpu.make_async_copy(
            in_hbm.at[pl.ds(row0, RCHUNK)], gins[p], isems[p]).wait()

    def wait_out(p):
        pltpu.make_async_copy(
            gouts[p], out_hbm.at[pl.ds(row0, RCHUNK)], osems[p]).wait()

    for g in range(DEPTH):
        pltpu.async_copy(
            in_hbm.at[pl.ds(row0 + g * RCHUNK, RCHUNK)], gins[g], isems[g])
    for g in range(DEPTH):
        wait_in(g)
        permute_chunk(gins[g], gouts[g])
        pltpu.async_copy(
            gouts[g], out_hbm.at[pl.ds(row0 + g * RCHUNK, RCHUNK)], osems[g])
        pltpu.async_copy(
            in_hbm.at[pl.ds(row0 + (g + DEPTH) * RCHUNK, RCHUNK)],
            gins[g], isems[g])

    def ring_body(gq, carry):
        for p in range(DEPTH):
            g = DEPTH * gq + p
            wait_in(p)
            wait_out(p)
            permute_chunk(gins[p], gouts[p])
            pltpu.async_copy(
                gouts[p], out_hbm.at[pl.ds(row0 + g * RCHUNK, RCHUNK)],
                osems[p])
            pltpu.async_copy(
                in_hbm.at[pl.ds(row0 + (g + DEPTH) * RCHUNK, RCHUNK)],
                gins[p], isems[p])
        return carry
    lax.fori_loop(1, NCHUNK // DEPTH - 1, ring_body, jnp.int32(0))

    for g in range(NCHUNK - DEPTH, NCHUNK):
        p = g % DEPTH
        wait_in(p)
        wait_out(p)
        permute_chunk(gins[p], gouts[p])
        pltpu.async_copy(
            gouts[p], out_hbm.at[pl.ds(row0 + g * RCHUNK, RCHUNK)], osems[p])
    for p in range(DEPTH):
        wait_out(p)


def kernel(inputs, indices):
    b, c, h, w = inputs.shape
    tbl = jnp.transpose(inputs, (0, 2, 3, 1)).reshape(b * h * w, c)
    mesh = plsc.VectorSubcoreMesh(core_axis_name="c", subcore_axis_name="s")
    run = functools.partial(
        pl.kernel,
        mesh=mesh,
        out_type=jax.ShapeDtypeStruct((b * h * w, c), jnp.float32),
        scratch_types=[
            pltpu.VMEM((C,), jnp.float32),
            pltpu.VMEM((C,), jnp.int32),
            pltpu.VMEM((RCHUNK, C), jnp.float32),
            pltpu.VMEM((RCHUNK, C), jnp.float32),
            pltpu.VMEM((RCHUNK, C), jnp.float32),
            pltpu.VMEM((RCHUNK, C), jnp.float32),
            pltpu.VMEM((RCHUNK, C), jnp.float32),
            pltpu.VMEM((RCHUNK, C), jnp.float32),
            pltpu.VMEM((RCHUNK, C), jnp.float32),
            pltpu.VMEM((RCHUNK, C), jnp.float32),
            pltpu.SemaphoreType.DMA,
            pltpu.SemaphoreType.DMA,
            pltpu.SemaphoreType.DMA,
            pltpu.SemaphoreType.DMA,
            pltpu.SemaphoreType.DMA,
            pltpu.SemaphoreType.DMA,
            pltpu.SemaphoreType.DMA,
            pltpu.SemaphoreType.DMA,
        ],
        compiler_params=pltpu.CompilerParams(
            use_tc_tiling_on_sc=True, needs_layout_passes=False),
    )(_sel_body)
    out = run(tbl, indices)
    return jnp.transpose(out.reshape(b, h, w, c), (0, 3, 1, 2))


# speedup vs baseline: 1.2520x; 1.2520x over previous
import functools

import jax
import jax.numpy as jnp
from jax import lax
from jax.experimental import pallas as pl
from jax.experimental.pallas import tpu as pltpu
from jax.experimental.pallas import tpu_sc as plsc

B = 32
C = 384
H = 56
W = 56
NROW = B * H * W
NW = 32
RPW = NROW // NW
RCHUNK = 64
NCHUNK = RPW // RCHUNK
L = 16
NJ = C // L


def _sel_body(in_hbm, ind_hbm, out_hbm, ind_v, idx_v,
              gin0, gin1, gout0, gout1, isem0, isem1, osem0, osem1):
    cid = lax.axis_index("c")
    sid = lax.axis_index("s")
    wid = sid * 2 + cid
    row0 = wid * RPW

    pltpu.sync_copy(ind_hbm, ind_v)

    zero_vec = jnp.zeros((L,), dtype=jnp.int32)
    for g in range(NJ):
        idx_v[pl.ds(g * L, L)] = zero_vec

    lane = lax.iota(jnp.int32, L)
    off = jnp.int32(0)
    for g in range(NJ):
        v = ind_v[pl.ds(g * L, L)]
        m = v != 0.0
        ind32 = jnp.where(m, 1, 0).astype(jnp.int32)
        cs = jnp.cumsum(ind32)
        pos = jnp.full((L,), off, dtype=jnp.int32) + cs - ind32
        vals = lane + (g * L)
        plsc.store_scatter(idx_v, [pos], vals, mask=m)
        off = off + jnp.max(cs)

    sel = [idx_v[pl.ds(j * L, L)] for j in range(NJ)]

    gins = (gin0, gin1)
    gouts = (gout0, gout1)
    isems = (isem0, isem1)
    osems = (osem0, osem1)

    def permute_chunk(gin, gout):
        @plsc.parallel_loop(0, RCHUNK, step=1, unroll=2)
        def body(r):
            row_splat = jnp.full((L,), r, dtype=jnp.int32)
            for j in range(NJ):
                vals = plsc.load_gather(gin, [row_splat, sel[j]])
                gout[r, pl.ds(j * L, L)] = vals

    def wait_in(p):
        pltpu.make_async_copy(
            in_hbm.at[pl.ds(row0, RCHUNK)], gins[p], isems[p]).wait()

    def wait_out(p):
        pltpu.make_async_copy(
            gouts[p], out_hbm.at[pl.ds(row0, RCHUNK)], osems[p]).wait()

    for g in range(2):
        pltpu.async_copy(
            in_hbm.at[pl.ds(row0 + g * RCHUNK, RCHUNK)], gins[g], isems[g])
    for g in range(2):
        wait_in(g)
        permute_chunk(gins[g], gouts[g])
        pltpu.async_copy(
            gouts[g], out_hbm.at[pl.ds(row0 + g * RCHUNK, RCHUNK)], osems[g])
        pltpu.async_copy(
            in_hbm.at[pl.ds(row0 + (g + 2) * RCHUNK, RCHUNK)],
            gins[g], isems[g])

    def pair_body(g2, carry):
        for p in range(2):
            g = 2 * g2 + p
            wait_in(p)
            wait_out(p)
            permute_chunk(gins[p], gouts[p])
            pltpu.async_copy(
                gouts[p], out_hbm.at[pl.ds(row0 + g * RCHUNK, RCHUNK)],
                osems[p])
            pltpu.async_copy(
                in_hbm.at[pl.ds(row0 + (g + 2) * RCHUNK, RCHUNK)],
                gins[p], isems[p])
        return carry
    lax.fori_loop(1, (NCHUNK - 3) // 2, pair_body, jnp.int32(0))

    for g in range(NCHUNK - 3, NCHUNK):
        p = g % 2
        wait_in(p)
        wait_out(p)
        permute_chunk(gins[p], gouts[p])
        pltpu.async_copy(
            gouts[p], out_hbm.at[pl.ds(row0 + g * RCHUNK, RCHUNK)], osems[p])
        if g + 2 < NCHUNK:
            pltpu.async_copy(
                in_hbm.at[pl.ds(row0 + (g + 2) * RCHUNK, RCHUNK)],
                gins[p], isems[p])
    wait_out(0)
    wait_out(1)


def kernel(inputs, indices):
    b, c, h, w = inputs.shape
    tbl = jnp.transpose(inputs, (0, 2, 3, 1)).reshape(b * h * w, c)
    mesh = plsc.VectorSubcoreMesh(core_axis_name="c", subcore_axis_name="s")
    run = functools.partial(
        pl.kernel,
        mesh=mesh,
        out_type=jax.ShapeDtypeStruct((b * h * w, c), jnp.float32),
        scratch_types=[
            pltpu.VMEM((C,), jnp.float32),
            pltpu.VMEM((C,), jnp.int32),
            pltpu.VMEM((RCHUNK, C), jnp.float32),
            pltpu.VMEM((RCHUNK, C), jnp.float32),
            pltpu.VMEM((RCHUNK, C), jnp.float32),
            pltpu.VMEM((RCHUNK, C), jnp.float32),
            pltpu.SemaphoreType.DMA,
            pltpu.SemaphoreType.DMA,
            pltpu.SemaphoreType.DMA,
            pltpu.SemaphoreType.DMA,
        ],
        compiler_params=pltpu.CompilerParams(
            use_tc_tiling_on_sc=True, needs_layout_passes=False),
    )(_sel_body)
    out = run(tbl, indices)
    return jnp.transpose(out.reshape(b, h, w, c), (0, 3, 1, 2))
